# baseline (speedup 1.0000x reference)
import functools

import jax
import jax.numpy as jnp
from jax import lax
from jax.experimental import pallas as pl
from jax.experimental.pallas import tpu as pltpu
from jax.experimental.pallas import tpu_sc as plsc

N_NODES = 50000
N_GRAPHS = 512
NP = 50176
E = 800000
EP = 819200
NIDXROWS = EP // 128

_MESH = plsc.VectorSubcoreMesh(core_axis_name="c", subcore_axis_name="s",
                               num_cores=2, num_subcores=16)


def _agg_pipeline(table, src_hbm, dst_hbm, acc, idxS, idxD, rows,
                  gsems, ssems, isem, base, nsuper):
    def rslice(b):
        return rows.at[pl.ds(b * 128, 128)]

    def drain16(sem, b):
        pltpu.make_async_copy(table.at[pl.ds(0, 128)], rslice(b), sem).wait()

    def drain_idx(buf):
        pltpu.make_async_copy(src_hbm.at[pl.ds(0, 8)], buf, isem).wait()

    def superblock(kb, k, first, last):
        if not first:
            drain_idx(idxS.at[kb])
            drain_idx(idxD.at[kb])
        for r in range(8):
            b = r % 4
            if not (first and r < 4):
                drain16(ssems[b], b)
            pltpu.async_copy(table.at[idxS.at[kb, r]], rslice(b), gsems[b])
            if not (first and r == 0):
                pr = (r - 1) % 8
                pkb = kb if r >= 1 else 1 - kb
                pb = pr % 4
                drain16(gsems[pb], pb)
                pltpu.async_copy(rslice(pb), acc.at[idxD.at[pkb, pr]],
                                 ssems[pb], add=True)
            if r == 4 and not last:
                nr0 = base + (k + 1) * 8
                pltpu.async_copy(src_hbm.at[pl.ds(nr0, 8)],
                                 idxS.at[1 - kb], isem)
                pltpu.async_copy(dst_hbm.at[pl.ds(nr0, 8)],
                                 idxD.at[1 - kb], isem)

    pltpu.sync_copy(src_hbm.at[pl.ds(base, 8)], idxS.at[0])
    pltpu.sync_copy(dst_hbm.at[pl.ds(base, 8)], idxD.at[0])
    superblock(0, 0, True, False)

    npairs = (nsuper - 2) // 2

    @pl.loop(0, npairs)
    def _steady(t):
        k1 = 1 + 2 * t
        superblock(1, k1, False, False)
        superblock(0, k1 + 1, False, False)

    for k in range(1 + 2 * npairs, nsuper):
        superblock(k % 2, k, False, k == nsuper - 1)

    lkb = (nsuper - 1) % 2
    drain16(gsems[3], 3)
    pltpu.async_copy(rslice(3), acc.at[idxD.at[lkb, 7]], ssems[3], add=True)
    for b in range(4):
        drain16(ssems[b], b)


def _agg_feat_body(h_hbm, src_hbm, dst_hbm, out_hbm,
                   acc, idxS, idxD, rows,
                   g0, g1, g2, g3, s0, s1, s2, s3, isem):
    c = lax.axis_index("c")
    s = lax.axis_index("s")

    for j in range(4):
        pltpu.sync_copy(h_hbm.at[c].at[pl.ds(s * 3136 + j * 784, 784)],
                        acc.at[pl.ds(s * 3136 + j * 784, 784)])
    plsc.subcore_barrier()

    _agg_pipeline(h_hbm.at[c], src_hbm, dst_hbm, acc, idxS, idxD, rows,
                  (g0, g1, g2, g3), (s0, s1, s2, s3), isem,
                  base=s * (NIDXROWS // 16), nsuper=NIDXROWS // 16 // 8)

    plsc.subcore_barrier()
    for j in range(4):
        pltpu.sync_copy(acc.at[pl.ds(s * 3136 + j * 784, 784)],
                        out_hbm.at[c].at[pl.ds(s * 3136 + j * 784, 784)])


@functools.partial(
    pl.kernel,
    out_type=jax.ShapeDtypeStruct((2, NP, 32), jnp.float32),
    mesh=_MESH,
    compiler_params=pltpu.CompilerParams(use_tc_tiling_on_sc=False),
    scratch_types=[
        pltpu.VMEM_SHARED((NP, 32), jnp.float32),
        pltpu.VMEM((2, 8, 128), jnp.int32),
        pltpu.VMEM((2, 8, 128), jnp.int32),
        pltpu.VMEM((512, 32), jnp.float32),
        pltpu.SemaphoreType.DMA,
        pltpu.SemaphoreType.DMA,
        pltpu.SemaphoreType.DMA,
        pltpu.SemaphoreType.DMA,
        pltpu.SemaphoreType.DMA,
        pltpu.SemaphoreType.DMA,
        pltpu.SemaphoreType.DMA,
        pltpu.SemaphoreType.DMA,
        pltpu.SemaphoreType.DMA,
    ],
)
def _sc_agg_feat(h_hbm, src_hbm, dst_hbm, out_hbm, *scratch):
    _agg_feat_body(h_hbm, src_hbm, dst_hbm, out_hbm, *scratch)


def _agg_edge_body(x_hbm, src_hbm, dst_hbm, out_hbm,
                   acc, idxS, idxD, rows, zbuf,
                   g0, g1, g2, g3, s0, s1, s2, s3, isem):
    c = lax.axis_index("c")
    s = lax.axis_index("s")

    @pl.when(c == 0)
    def _():
        for j in range(4):
            pltpu.sync_copy(x_hbm.at[pl.ds(s * 3136 + j * 784, 784)],
                            acc.at[pl.ds(s * 3136 + j * 784, 784)])

    @pl.when(c == 1)
    def _():
        @pl.loop(0, 784)
        def _zero(i):
            zbuf[i, pl.ds(0, 16)] = jnp.zeros((16,), jnp.float32)

        for j in range(4):
            pltpu.sync_copy(zbuf, acc.at[pl.ds(s * 3136 + j * 784, 784)])

    plsc.subcore_barrier()

    w = s * 2 + c
    _agg_pipeline(x_hbm, src_hbm, dst_hbm, acc, idxS, idxD, rows,
                  (g0, g1, g2, g3), (s0, s1, s2, s3), isem,
                  base=w * (NIDXROWS // 32), nsuper=NIDXROWS // 32 // 8)

    plsc.subcore_barrier()
    for j in range(4):
        pltpu.sync_copy(acc.at[pl.ds(s * 3136 + j * 784, 784)],
                        out_hbm.at[c].at[pl.ds(s * 3136 + j * 784, 784)])


@functools.partial(
    pl.kernel,
    out_type=jax.ShapeDtypeStruct((2, NP, 16), jnp.float32),
    mesh=_MESH,
    compiler_params=pltpu.CompilerParams(use_tc_tiling_on_sc=False),
    scratch_types=[
        pltpu.VMEM_SHARED((NP, 16), jnp.float32),
        pltpu.VMEM((2, 8, 128), jnp.int32),
        pltpu.VMEM((2, 8, 128), jnp.int32),
        pltpu.VMEM((512, 16), jnp.float32),
        pltpu.VMEM((784, 16), jnp.float32),
        pltpu.SemaphoreType.DMA,
        pltpu.SemaphoreType.DMA,
        pltpu.SemaphoreType.DMA,
        pltpu.SemaphoreType.DMA,
        pltpu.SemaphoreType.DMA,
        pltpu.SemaphoreType.DMA,
        pltpu.SemaphoreType.DMA,
        pltpu.SemaphoreType.DMA,
        pltpu.SemaphoreType.DMA,
    ],
)
def _sc_agg_edge(x_hbm, src_hbm, dst_hbm, out_hbm, *scratch):
    _agg_edge_body(x_hbm, src_hbm, dst_hbm, out_hbm, *scratch)


NPG = 528
BROWS = NP // 128


def _pool_body(h_hbm, bpad_hbm, out_hbm,
               psum, pcnt, stage, pmax, hbuf, ones, bidx, zb32, zb16,
               bsmem, tbuf, sbuf, cbuf, obuf, gsem):
    c = lax.axis_index("c")
    s = lax.axis_index("s")
    NEG = jnp.float32(-jnp.inf)

    @pl.loop(0, NPG)
    def _initmax(i):
        pmax[i, pl.ds(0, 16)] = jnp.full((16,), NEG, jnp.float32)
        pmax[i, pl.ds(16, 16)] = jnp.full((16,), NEG, jnp.float32)

    @pl.loop(0, 128)
    def _initones(i):
        ones[i, pl.ds(0, 16)] = jnp.ones((16,), jnp.float32)

    @pl.loop(0, 33)
    def _initz(i):
        zb32[i, pl.ds(0, 16)] = jnp.zeros((16,), jnp.float32)
        zb32[i, pl.ds(16, 16)] = jnp.zeros((16,), jnp.float32)
        zb16[i, pl.ds(0, 16)] = jnp.zeros((16,), jnp.float32)

    pltpu.sync_copy(zb32, psum.at[pl.ds(s * 33, 33)])
    pltpu.sync_copy(zb16, pcnt.at[pl.ds(s * 33, 33)])
    plsc.subcore_barrier()

    @pl.loop(0, 25)
    def _sums(t):
        j = s + 16 * t

        @pl.when(j < BROWS)
        def _():
            pltpu.sync_copy(bpad_hbm.at[pl.ds(j * 128, 128)], bidx)
            pltpu.sync_copy(h_hbm.at[c].at[pl.ds(j * 128, 128)],
                            hbuf.at[pl.ds(0, 128)])
            pltpu.sync_copy(hbuf.at[pl.ds(0, 128)],
                            psum.at[bidx], add=True)
            pltpu.sync_copy(ones, pcnt.at[bidx], add=True)

    for t in range(14):
        r0 = s * 3136 + t * 224
        pltpu.sync_copy(h_hbm.at[c].at[pl.ds(r0, 224)], hbuf.at[pl.ds(0, 224)])
        pltpu.sync_copy(bpad_hbm.at[pl.ds(r0, 224)], bsmem)

        @pl.loop(0, 14)
        def _grp(tg):
            base_r = tg * 16
            gvec = bsmem[pl.ds(base_r, 16)]
            for i in range(16):
                g = gvec[i]
                r = base_r + i
                v0 = hbuf[r, pl.ds(0, 16)]
                v1 = hbuf[r, pl.ds(16, 16)]
                pmax[g, pl.ds(0, 16)] = jnp.maximum(pmax[g, pl.ds(0, 16)], v0)
                pmax[g, pl.ds(16, 16)] = jnp.maximum(pmax[g, pl.ds(16, 16)], v1)

    pltpu.sync_copy(pmax.at[pl.ds(0, 512)], stage.at[s])
    plsc.subcore_barrier()

    g0 = s * 32
    pltpu.sync_copy(stage.at[:, pl.ds(g0, 32), :], tbuf)
    pltpu.sync_copy(psum.at[pl.ds(g0, 32)], sbuf)
    pltpu.sync_copy(pcnt.at[pl.ds(g0, 32)], cbuf)

    @pl.loop(0, 32)
    def _comb(i):
        m0 = tbuf[0, i, pl.ds(0, 16)]
        m1 = tbuf[0, i, pl.ds(16, 16)]
        for k in range(1, 16):
            m0 = jnp.maximum(m0, tbuf[k, i, pl.ds(0, 16)])
            m1 = jnp.maximum(m1, tbuf[k, i, pl.ds(16, 16)])
        cnt = jnp.maximum(cbuf[i, pl.ds(0, 16)], 1.0)
        obuf[i, pl.ds(0, 16)] = sbuf[i, pl.ds(0, 16)] / cnt
        obuf[i, pl.ds(16, 16)] = sbuf[i, pl.ds(16, 16)] / cnt
        hbuf[i, pl.ds(0, 16)] = m0
        hbuf[i, pl.ds(16, 16)] = m1

    pltpu.sync_copy(obuf, out_hbm.at[c].at[pl.ds(g0, 32)])
    pltpu.sync_copy(hbuf.at[pl.ds(0, 32)], out_hbm.at[2 + c].at[pl.ds(g0, 32)])


@functools.partial(
    pl.kernel,
    out_type=jax.ShapeDtypeStruct((4, 512, 32), jnp.float32),
    mesh=_MESH,
    compiler_params=pltpu.CompilerParams(use_tc_tiling_on_sc=False),
    scratch_types=[
        pltpu.VMEM_SHARED((NPG, 32), jnp.float32),
        pltpu.VMEM_SHARED((NPG, 16), jnp.float32),
        pltpu.VMEM_SHARED((16, 512, 32), jnp.float32),
        pltpu.VMEM((NPG, 32), jnp.float32),
        pltpu.VMEM((224, 32), jnp.float32),
        pltpu.VMEM((128, 16), jnp.float32),
        pltpu.VMEM((128,), jnp.int32),
        pltpu.VMEM((33, 32), jnp.float32),
        pltpu.VMEM((33, 16), jnp.float32),
        pltpu.VMEM((224,), jnp.int32),
        pltpu.VMEM((16, 32, 32), jnp.float32),
        pltpu.VMEM((32, 32), jnp.float32),
        pltpu.VMEM((32, 16), jnp.float32),
        pltpu.VMEM((32, 32), jnp.float32),
        pltpu.SemaphoreType.DMA,
    ],
)
def _sc_pool(h_hbm, bpad_hbm, out_hbm, *scratch):
    _pool_body(h_hbm, bpad_hbm, out_hbm, *scratch)


_BR = 3584
_NBLK = NP // _BR


def _layer_common(hb, i, w1_ref, b1_ref, g_ref, bt_ref, w2_ref, b2_ref,
                  out_ref, ssum, ssq, coef):
    p = pl.program_id(0)
    h1 = jnp.dot(hb, w1_ref[...].T,
                 preferred_element_type=jnp.float32) + b1_ref[...]
    rows = i * _BR + lax.broadcasted_iota(jnp.int32, (_BR, 1), 0)
    h1 = jnp.where(rows < N_NODES, h1, 0.0)

    @pl.when(p == 0)
    def _():
        @pl.when(i == 0)
        def _():
            ssum[...] = jnp.zeros_like(ssum)
            ssq[...] = jnp.zeros_like(ssq)

        ssum[...] += jnp.sum(h1, axis=0, keepdims=True)
        ssq[...] += jnp.sum(h1 * h1, axis=0, keepdims=True)

    @pl.when(p == 1)
    def _():
        @pl.when(i == 0)
        def _():
            m = ssum[...] / N_NODES
            v = ssq[...] / N_NODES - m * m
            sc = g_ref[...] / jnp.sqrt(v + 1e-5)
            coef[0:1] = sc
            coef[1:2] = bt_ref[...] - m * sc

        h = jnp.maximum(h1 * coef[0:1] + coef[1:2], 0.0)
        h2 = jnp.maximum(
            jnp.dot(h, w2_ref[...].T, preferred_element_type=jnp.float32)
            + b2_ref[...], 0.0)
        h2 = jnp.where(rows < N_NODES, h2, 0.0)
        out_ref[0] = h2[:, :32]
        out_ref[1] = h2[:, 32:]


def _layer64_body(agg_ref, w1_ref, b1_ref, g_ref, bt_ref,
                  w2_ref, b2_ref, out_ref, ssum, ssq, coef):
    i = pl.program_id(1)
    hb = jnp.concatenate([agg_ref[0], agg_ref[1]], axis=1)
    _layer_common(hb, i, w1_ref, b1_ref, g_ref, bt_ref, w2_ref, b2_ref,
                  out_ref, ssum, ssq, coef)


def _layer16_body(agg_ref, w1_ref, b1_ref, g_ref, bt_ref,
                  w2_ref, b2_ref, out_ref, ssum, ssq, coef):
    i = pl.program_id(1)
    hb = agg_ref[0] + agg_ref[1]
    _layer_common(hb, i, w1_ref, b1_ref, g_ref, bt_ref, w2_ref, b2_ref,
                  out_ref, ssum, ssq, coef)


def _tc_layer(body, agg, width, W1, b1, g, bt, W2, b2):
    return pl.pallas_call(
        body,
        grid=(2, _NBLK),
        in_specs=[
            pl.BlockSpec((2, _BR, width), lambda p, i: (0, i, 0)),
            pl.BlockSpec((64, 64 if width == 32 else 16),
                         lambda p, i: (0, 0)),
            pl.BlockSpec((1, 64), lambda p, i: (0, 0)),
            pl.BlockSpec((1, 64), lambda p, i: (0, 0)),
            pl.BlockSpec((1, 64), lambda p, i: (0, 0)),
            pl.BlockSpec((64, 64), lambda p, i: (0, 0)),
            pl.BlockSpec((1, 64), lambda p, i: (0, 0)),
        ],
        out_specs=pl.BlockSpec((2, _BR, 32), lambda p, i: (0, i, 0)),
        out_shape=jax.ShapeDtypeStruct((2, NP, 32), jnp.float32),
        scratch_shapes=[
            pltpu.VMEM((1, 64), jnp.float32),
            pltpu.VMEM((1, 64), jnp.float32),
            pltpu.VMEM((2, 64), jnp.float32),
        ],
    )(agg, W1, b1.reshape(1, -1), g.reshape(1, -1), bt.reshape(1, -1),
      W2, b2.reshape(1, -1))


def _cls_body(pooled_ref, w1_ref, b1_ref, w2_ref, b2_ref, out_ref):
    z = jnp.maximum(
        jnp.dot(pooled_ref[...], w1_ref[...].T,
                preferred_element_type=jnp.float32) + b1_ref[...], 0.0)
    out_ref[...] = (
        jnp.dot(z, w2_ref[...].T, preferred_element_type=jnp.float32)
        + b2_ref[...])


def _classifier(pooled, w1, b1, w2, b2):
    return pl.pallas_call(
        _cls_body,
        out_shape=jax.ShapeDtypeStruct((N_GRAPHS, w2.shape[0]), jnp.float32),
    )(pooled, w1, b1.reshape(1, -1), w2, b2.reshape(1, -1))


def kernel(x, edge_index, batch, c1_W1, c1_b1, c1_g, c1_bt, c1_W2, c1_b2,
           c2_W1, c2_b1, c2_g, c2_bt, c2_W2, c2_b2,
           c3_W1, c3_b1, c3_g, c3_bt, c3_W2, c3_b2,
           cls_W1, cls_b1, cls_W2, cls_b2):
    src = jnp.concatenate(
        [edge_index[0], jnp.full((EP - E,), N_NODES, jnp.int32)]
    ).reshape(NIDXROWS, 128)
    dst = jnp.concatenate(
        [edge_index[1],
         N_NODES + (jnp.arange(EP - E, dtype=jnp.int32) % (NP - N_NODES))]
    ).reshape(NIDXROWS, 128)

    xp = jnp.pad(x, ((0, NP - N_NODES), (0, 6)))
    W1p = jnp.pad(c1_W1, ((0, 0), (0, 6)))
    agg1 = _sc_agg_edge(xp, src, dst)
    hs = _tc_layer(_layer16_body, agg1, 16,
                   W1p, c1_b1, c1_g, c1_bt, c1_W2, c1_b2)

    agg2 = _sc_agg_feat(hs, src, dst)
    hs = _tc_layer(_layer64_body, agg2, 32,
                   c2_W1, c2_b1, c2_g, c2_bt, c2_W2, c2_b2)

    agg3 = _sc_agg_feat(hs, src, dst)
    hs = _tc_layer(_layer64_body, agg3, 32,
                   c3_W1, c3_b1, c3_g, c3_bt, c3_W2, c3_b2)

    bpad = jnp.concatenate(
        [batch, jnp.full((NP - N_NODES,), N_GRAPHS, jnp.int32)])
    pooled4 = _sc_pool(hs, bpad)
    pooled = jnp.concatenate(
        [pooled4[0], pooled4[1], pooled4[2], pooled4[3]], axis=1)
    return _classifier(pooled, cls_W1, cls_b1, cls_W2, cls_b2)


# speedup vs baseline: 1.5304x; 1.5304x over previous
import functools

import jax
import jax.numpy as jnp
from jax import lax
from jax.experimental import pallas as pl
from jax.experimental.pallas import tpu as pltpu
from jax.experimental.pallas import tpu_sc as plsc

N_NODES = 50000
N_GRAPHS = 512
NP = 50176
E = 800000
EP = 819200
NIDXROWS = EP // 128

_MESH = plsc.VectorSubcoreMesh(core_axis_name="c", subcore_axis_name="s",
                               num_cores=2, num_subcores=16)


def _agg_pipeline(table, src_hbm, dst_hbm, acc, idxS, idxD, rows,
                  gsems, ssems, isem, base, nsuper):
    def rslice(b):
        return rows.at[pl.ds(b * 128, 128)]

    def drain16(sem, b):
        pltpu.make_async_copy(table.at[pl.ds(0, 128)], rslice(b), sem).wait()

    def drain_idx(buf):
        pltpu.make_async_copy(src_hbm.at[pl.ds(0, 8)], buf, isem).wait()

    def superblock(kb, k, first, last):
        if not first:
            drain_idx(idxS.at[kb])
            drain_idx(idxD.at[kb])
        for r in range(8):
            b = r % 4
            if not (first and r < 4):
                drain16(ssems[b], b)
            pltpu.async_copy(table.at[idxS.at[kb, r]], rslice(b), gsems[b])
            if not (first and r == 0):
                pr = (r - 1) % 8
                pkb = kb if r >= 1 else 1 - kb
                pb = pr % 4
                drain16(gsems[pb], pb)
                pltpu.async_copy(rslice(pb), acc.at[idxD.at[pkb, pr]],
                                 ssems[pb], add=True)
            if r == 4 and not last:
                nr0 = base + (k + 1) * 8
                pltpu.async_copy(src_hbm.at[pl.ds(nr0, 8)],
                                 idxS.at[1 - kb], isem)
                pltpu.async_copy(dst_hbm.at[pl.ds(nr0, 8)],
                                 idxD.at[1 - kb], isem)

    pltpu.sync_copy(src_hbm.at[pl.ds(base, 8)], idxS.at[0])
    pltpu.sync_copy(dst_hbm.at[pl.ds(base, 8)], idxD.at[0])
    superblock(0, 0, True, False)

    npairs = (nsuper - 2) // 2

    @pl.loop(0, npairs)
    def _steady(t):
        k1 = 1 + 2 * t
        superblock(1, k1, False, False)
        superblock(0, k1 + 1, False, False)

    for k in range(1 + 2 * npairs, nsuper):
        superblock(k % 2, k, False, k == nsuper - 1)

    lkb = (nsuper - 1) % 2
    drain16(gsems[3], 3)
    pltpu.async_copy(rslice(3), acc.at[idxD.at[lkb, 7]], ssems[3], add=True)
    for b in range(4):
        drain16(ssems[b], b)


def _agg_feat_body(h_hbm, src_hbm, dst_hbm, out_hbm,
                   acc, idxS, idxD, rows,
                   g0, g1, g2, g3, s0, s1, s2, s3, isem):
    c = lax.axis_index("c")
    s = lax.axis_index("s")

    for j in range(4):
        pltpu.sync_copy(h_hbm.at[c].at[pl.ds(s * 3136 + j * 784, 784)],
                        acc.at[pl.ds(s * 3136 + j * 784, 784)])
    plsc.subcore_barrier()

    _agg_pipeline(h_hbm.at[c], src_hbm, dst_hbm, acc, idxS, idxD, rows,
                  (g0, g1, g2, g3), (s0, s1, s2, s3), isem,
                  base=s * (NIDXROWS // 16), nsuper=NIDXROWS // 16 // 8)

    plsc.subcore_barrier()
    for j in range(4):
        pltpu.sync_copy(acc.at[pl.ds(s * 3136 + j * 784, 784)],
                        out_hbm.at[c].at[pl.ds(s * 3136 + j * 784, 784)])


@functools.partial(
    pl.kernel,
    out_type=jax.ShapeDtypeStruct((2, NP, 32), jnp.float32),
    mesh=_MESH,
    compiler_params=pltpu.CompilerParams(use_tc_tiling_on_sc=False),
    scratch_types=[
        pltpu.VMEM_SHARED((NP, 32), jnp.float32),
        pltpu.VMEM((2, 8, 128), jnp.int32),
        pltpu.VMEM((2, 8, 128), jnp.int32),
        pltpu.VMEM((512, 32), jnp.float32),
        pltpu.SemaphoreType.DMA,
        pltpu.SemaphoreType.DMA,
        pltpu.SemaphoreType.DMA,
        pltpu.SemaphoreType.DMA,
        pltpu.SemaphoreType.DMA,
        pltpu.SemaphoreType.DMA,
        pltpu.SemaphoreType.DMA,
        pltpu.SemaphoreType.DMA,
        pltpu.SemaphoreType.DMA,
    ],
)
def _sc_agg_feat(h_hbm, src_hbm, dst_hbm, out_hbm, *scratch):
    _agg_feat_body(h_hbm, src_hbm, dst_hbm, out_hbm, *scratch)


def _agg_edge_body(x_hbm, src_hbm, dst_hbm, out_hbm,
                   acc, idxS, idxD, rows, zbuf,
                   g0, g1, g2, g3, s0, s1, s2, s3, isem):
    c = lax.axis_index("c")
    s = lax.axis_index("s")

    @pl.when(c == 0)
    def _():
        for j in range(4):
            pltpu.sync_copy(x_hbm.at[pl.ds(s * 3136 + j * 784, 784)],
                            acc.at[pl.ds(s * 3136 + j * 784, 784)])

    @pl.when(c == 1)
    def _():
        @pl.loop(0, 784)
        def _zero(i):
            zbuf[i, pl.ds(0, 16)] = jnp.zeros((16,), jnp.float32)

        for j in range(4):
            pltpu.sync_copy(zbuf, acc.at[pl.ds(s * 3136 + j * 784, 784)])

    plsc.subcore_barrier()

    w = s * 2 + c
    _agg_pipeline(x_hbm, src_hbm, dst_hbm, acc, idxS, idxD, rows,
                  (g0, g1, g2, g3), (s0, s1, s2, s3), isem,
                  base=w * (NIDXROWS // 32), nsuper=NIDXROWS // 32 // 8)

    plsc.subcore_barrier()
    for j in range(4):
        pltpu.sync_copy(acc.at[pl.ds(s * 3136 + j * 784, 784)],
                        out_hbm.at[c].at[pl.ds(s * 3136 + j * 784, 784)])


@functools.partial(
    pl.kernel,
    out_type=jax.ShapeDtypeStruct((2, NP, 16), jnp.float32),
    mesh=_MESH,
    compiler_params=pltpu.CompilerParams(use_tc_tiling_on_sc=False),
    scratch_types=[
        pltpu.VMEM_SHARED((NP, 16), jnp.float32),
        pltpu.VMEM((2, 8, 128), jnp.int32),
        pltpu.VMEM((2, 8, 128), jnp.int32),
        pltpu.VMEM((512, 16), jnp.float32),
        pltpu.VMEM((784, 16), jnp.float32),
        pltpu.SemaphoreType.DMA,
        pltpu.SemaphoreType.DMA,
        pltpu.SemaphoreType.DMA,
        pltpu.SemaphoreType.DMA,
        pltpu.SemaphoreType.DMA,
        pltpu.SemaphoreType.DMA,
        pltpu.SemaphoreType.DMA,
        pltpu.SemaphoreType.DMA,
        pltpu.SemaphoreType.DMA,
    ],
)
def _sc_agg_edge(x_hbm, src_hbm, dst_hbm, out_hbm, *scratch):
    _agg_edge_body(x_hbm, src_hbm, dst_hbm, out_hbm, *scratch)


NPG = 528
BROWS = NP // 128


def _pool_body(h_hbm, bpad_hbm, out_hbm,
               psum, pcnt, stage, pmax, hbuf, ones, bidx, zb32, zb16,
               bsmem, tbuf, sbuf, cbuf, obuf, gsem):
    c = lax.axis_index("c")
    s = lax.axis_index("s")
    NEG = jnp.float32(-jnp.inf)

    @pl.loop(0, NPG)
    def _initmax(i):
        pmax[i, pl.ds(0, 16)] = jnp.full((16,), NEG, jnp.float32)
        pmax[i, pl.ds(16, 16)] = jnp.full((16,), NEG, jnp.float32)

    @pl.loop(0, 128)
    def _initones(i):
        ones[i, pl.ds(0, 16)] = jnp.ones((16,), jnp.float32)

    @pl.loop(0, 33)
    def _initz(i):
        zb32[i, pl.ds(0, 16)] = jnp.zeros((16,), jnp.float32)
        zb32[i, pl.ds(16, 16)] = jnp.zeros((16,), jnp.float32)
        zb16[i, pl.ds(0, 16)] = jnp.zeros((16,), jnp.float32)

    pltpu.sync_copy(zb32, psum.at[pl.ds(s * 33, 33)])
    pltpu.sync_copy(zb16, pcnt.at[pl.ds(s * 33, 33)])
    plsc.subcore_barrier()

    @pl.loop(0, 25)
    def _sums(t):
        j = s + 16 * t

        @pl.when(j < BROWS)
        def _():
            pltpu.sync_copy(bpad_hbm.at[pl.ds(j * 128, 128)], bidx)
            pltpu.sync_copy(h_hbm.at[c].at[pl.ds(j * 128, 128)],
                            hbuf.at[pl.ds(0, 128)])
            pltpu.sync_copy(hbuf.at[pl.ds(0, 128)],
                            psum.at[bidx], add=True)
            pltpu.sync_copy(ones, pcnt.at[bidx], add=True)

    for t in range(14):
        r0 = s * 3136 + t * 224
        pltpu.sync_copy(h_hbm.at[c].at[pl.ds(r0, 224)], hbuf.at[pl.ds(0, 224)])
        pltpu.sync_copy(bpad_hbm.at[pl.ds(r0, 224)], bsmem)

        @pl.loop(0, 14)
        def _grp(tg):
            base_r = tg * 16
            gvec = bsmem[pl.ds(base_r, 16)]
            for i in range(16):
                g = gvec[i]
                r = base_r + i
                v0 = hbuf[r, pl.ds(0, 16)]
                v1 = hbuf[r, pl.ds(16, 16)]
                pmax[g, pl.ds(0, 16)] = jnp.maximum(pmax[g, pl.ds(0, 16)], v0)
                pmax[g, pl.ds(16, 16)] = jnp.maximum(pmax[g, pl.ds(16, 16)], v1)

    pltpu.sync_copy(pmax.at[pl.ds(0, 512)], stage.at[s])
    plsc.subcore_barrier()

    g0 = s * 32
    pltpu.sync_copy(stage.at[:, pl.ds(g0, 32), :], tbuf)
    pltpu.sync_copy(psum.at[pl.ds(g0, 32)], sbuf)
    pltpu.sync_copy(pcnt.at[pl.ds(g0, 32)], cbuf)

    @pl.loop(0, 32)
    def _comb(i):
        m0 = tbuf[0, i, pl.ds(0, 16)]
        m1 = tbuf[0, i, pl.ds(16, 16)]
        for k in range(1, 16):
            m0 = jnp.maximum(m0, tbuf[k, i, pl.ds(0, 16)])
            m1 = jnp.maximum(m1, tbuf[k, i, pl.ds(16, 16)])
        cnt = jnp.maximum(cbuf[i, pl.ds(0, 16)], 1.0)
        obuf[i, pl.ds(0, 16)] = sbuf[i, pl.ds(0, 16)] / cnt
        obuf[i, pl.ds(16, 16)] = sbuf[i, pl.ds(16, 16)] / cnt
        hbuf[i, pl.ds(0, 16)] = m0
        hbuf[i, pl.ds(16, 16)] = m1

    pltpu.sync_copy(obuf, out_hbm.at[c].at[pl.ds(g0, 32)])
    pltpu.sync_copy(hbuf.at[pl.ds(0, 32)], out_hbm.at[2 + c].at[pl.ds(g0, 32)])


@functools.partial(
    pl.kernel,
    out_type=jax.ShapeDtypeStruct((4, 512, 32), jnp.float32),
    mesh=_MESH,
    compiler_params=pltpu.CompilerParams(use_tc_tiling_on_sc=False),
    scratch_types=[
        pltpu.VMEM_SHARED((NPG, 32), jnp.float32),
        pltpu.VMEM_SHARED((NPG, 16), jnp.float32),
        pltpu.VMEM_SHARED((16, 512, 32), jnp.float32),
        pltpu.VMEM((NPG, 32), jnp.float32),
        pltpu.VMEM((224, 32), jnp.float32),
        pltpu.VMEM((128, 16), jnp.float32),
        pltpu.VMEM((128,), jnp.int32),
        pltpu.VMEM((33, 32), jnp.float32),
        pltpu.VMEM((33, 16), jnp.float32),
        pltpu.VMEM((224,), jnp.int32),
        pltpu.VMEM((16, 32, 32), jnp.float32),
        pltpu.VMEM((32, 32), jnp.float32),
        pltpu.VMEM((32, 16), jnp.float32),
        pltpu.VMEM((32, 32), jnp.float32),
        pltpu.SemaphoreType.DMA,
    ],
)
def _sc_pool(h_hbm, bpad_hbm, out_hbm, *scratch):
    _pool_body(h_hbm, bpad_hbm, out_hbm, *scratch)


_BR = 3584
_NBLK = NP // _BR


def _layer_common(hb, i, w1_ref, b1_ref, g_ref, bt_ref, w2_ref, b2_ref,
                  out_ref, ssum, ssq, coef):
    p = pl.program_id(0)
    h1 = jnp.dot(hb, w1_ref[...].T,
                 preferred_element_type=jnp.float32) + b1_ref[...]
    rows = i * _BR + lax.broadcasted_iota(jnp.int32, (_BR, 1), 0)
    h1 = jnp.where(rows < N_NODES, h1, 0.0)

    @pl.when(p == 0)
    def _():
        @pl.when(i == 0)
        def _():
            ssum[...] = jnp.zeros_like(ssum)
            ssq[...] = jnp.zeros_like(ssq)

        ssum[...] += jnp.sum(h1, axis=0, keepdims=True)
        ssq[...] += jnp.sum(h1 * h1, axis=0, keepdims=True)

    @pl.when(p == 1)
    def _():
        @pl.when(i == 0)
        def _():
            m = ssum[...] / N_NODES
            v = ssq[...] / N_NODES - m * m
            sc = g_ref[...] / jnp.sqrt(v + 1e-5)
            coef[0:1] = sc
            coef[1:2] = bt_ref[...] - m * sc

        h = jnp.maximum(h1 * coef[0:1] + coef[1:2], 0.0)
        h2 = jnp.maximum(
            jnp.dot(h, w2_ref[...].T, preferred_element_type=jnp.float32)
            + b2_ref[...], 0.0)
        h2 = jnp.where(rows < N_NODES, h2, 0.0)
        out_ref[0] = h2[:, :32]
        out_ref[1] = h2[:, 32:]


def _layer64_body(agg_ref, w1_ref, b1_ref, g_ref, bt_ref,
                  w2_ref, b2_ref, out_ref, ssum, ssq, coef):
    i = pl.program_id(1)
    hb = jnp.concatenate([agg_ref[0], agg_ref[1]], axis=1)
    _layer_common(hb, i, w1_ref, b1_ref, g_ref, bt_ref, w2_ref, b2_ref,
                  out_ref, ssum, ssq, coef)


def _layer16_body(agg_ref, w1_ref, b1_ref, g_ref, bt_ref,
                  w2_ref, b2_ref, out_ref, ssum, ssq, coef):
    i = pl.program_id(1)
    hb = agg_ref[0] + agg_ref[1]
    _layer_common(hb, i, w1_ref, b1_ref, g_ref, bt_ref, w2_ref, b2_ref,
                  out_ref, ssum, ssq, coef)


def _tc_layer(body, agg, width, W1, b1, g, bt, W2, b2):
    return pl.pallas_call(
        body,
        grid=(2, _NBLK),
        in_specs=[
            pl.BlockSpec((2, _BR, width), lambda p, i: (0, i, 0)),
            pl.BlockSpec((64, 64 if width == 32 else 16),
                         lambda p, i: (0, 0)),
            pl.BlockSpec((1, 64), lambda p, i: (0, 0)),
            pl.BlockSpec((1, 64), lambda p, i: (0, 0)),
            pl.BlockSpec((1, 64), lambda p, i: (0, 0)),
            pl.BlockSpec((64, 64), lambda p, i: (0, 0)),
            pl.BlockSpec((1, 64), lambda p, i: (0, 0)),
        ],
        out_specs=pl.BlockSpec((2, _BR, 32), lambda p, i: (0, i, 0)),
        out_shape=jax.ShapeDtypeStruct((2, NP, 32), jnp.float32),
        scratch_shapes=[
            pltpu.VMEM((1, 64), jnp.float32),
            pltpu.VMEM((1, 64), jnp.float32),
            pltpu.VMEM((2, 64), jnp.float32),
        ],
    )(agg, W1, b1.reshape(1, -1), g.reshape(1, -1), bt.reshape(1, -1),
      W2, b2.reshape(1, -1))


def _cls_body(pooled_ref, w1_ref, b1_ref, w2_ref, b2_ref, out_ref):
    z = jnp.maximum(
        jnp.dot(pooled_ref[...], w1_ref[...].T,
                preferred_element_type=jnp.float32) + b1_ref[...], 0.0)
    out_ref[...] = (
        jnp.dot(z, w2_ref[...].T, preferred_element_type=jnp.float32)
        + b2_ref[...])


def _classifier(pooled, w1, b1, w2, b2):
    return pl.pallas_call(
        _cls_body,
        out_shape=jax.ShapeDtypeStruct((N_GRAPHS, w2.shape[0]), jnp.float32),
    )(pooled, w1, b1.reshape(1, -1), w2, b2.reshape(1, -1))


def kernel(x, edge_index, batch, c1_W1, c1_b1, c1_g, c1_bt, c1_W2, c1_b2,
           c2_W1, c2_b1, c2_g, c2_bt, c2_W2, c2_b2,
           c3_W1, c3_b1, c3_g, c3_bt, c3_W2, c3_b2,
           cls_W1, cls_b1, cls_W2, cls_b2):
    src = jnp.concatenate(
        [edge_index[0],
         (jnp.arange(EP - E, dtype=jnp.int32) * 41) % N_NODES]
    ).reshape(NIDXROWS, 128)
    dst = jnp.concatenate(
        [edge_index[1],
         N_NODES + (jnp.arange(EP - E, dtype=jnp.int32) % (NP - N_NODES))]
    ).reshape(NIDXROWS, 128)

    xp = jnp.pad(x, ((0, NP - N_NODES), (0, 6)))
    W1p = jnp.pad(c1_W1, ((0, 0), (0, 6)))
    agg1 = _sc_agg_edge(xp, src, dst)
    hs = _tc_layer(_layer16_body, agg1, 16,
                   W1p, c1_b1, c1_g, c1_bt, c1_W2, c1_b2)

    agg2 = _sc_agg_feat(hs, src, dst)
    hs = _tc_layer(_layer64_body, agg2, 32,
                   c2_W1, c2_b1, c2_g, c2_bt, c2_W2, c2_b2)

    agg3 = _sc_agg_feat(hs, src, dst)
    hs = _tc_layer(_layer64_body, agg3, 32,
                   c3_W1, c3_b1, c3_g, c3_bt, c3_W2, c3_b2)

    bpad = jnp.concatenate(
        [batch, jnp.full((NP - N_NODES,), N_GRAPHS, jnp.int32)])
    pooled4 = _sc_pool(hs, bpad)
    pooled = jnp.concatenate(
        [pooled4[0], pooled4[1], pooled4[2], pooled4[3]], axis=1)
    return _classifier(pooled, cls_W1, cls_b1, cls_W2, cls_b2)
